# Initial kernel scaffold; baseline (speedup 1.0000x reference)
import jax
import jax.numpy as jnp
from jax.experimental import pallas as pl


def kernel(x, edge_index, enc_W, enc_b, ln_g, ln_b, W1, b1, W2, b2, W3, b3, ec_W1, ec_b1, ec_W2, ec_b2, ph_W1, ph_b1, ph_W2, ph_b2, ch_W1, ch_b1, ch_W2, ch_b2):
    raise NotImplementedError("write your pallas kernel here")


# speedup vs baseline: 4.1063x; 4.1063x over previous
import functools
import jax
import jax.numpy as jnp
from jax import lax
from jax.experimental import pallas as pl
from jax.experimental.pallas import tpu as pltpu
from jax.experimental.pallas import tpu_sc as plsc

N = 10000
E = 320000
H = 128
NC, NS = 2, 16
NW = NC * NS
EPC = E // NC
EPW = E // NW
CH = 80
NCHUNK = EPW // CH
ROWS_PW = 624
OWN = 312
NEG = -3.0e38

_mesh = plsc.VectorSubcoreMesh(core_axis_name="c", subcore_axis_name="s")
_sc_params = pltpu.CompilerParams(needs_layout_passes=False)


def _worker_edge_base(cid, sid):
    return cid * EPC + sid * EPW


def _deg_body(dst_hbm, out_hbm, acc_sh, ones_v, dst_v, zrow_v):
    cid = lax.axis_index("c")
    sid = lax.axis_index("s")

    def fill_body(i, _):
        for j in range(8):
            ones_v[i, pl.ds(j * 16, 16)] = jnp.ones((16,), jnp.float32)
            zrow_v[i, pl.ds(j * 16, 16)] = jnp.zeros((16,), jnp.float32)
        return _

    lax.fori_loop(0, CH, fill_body, jnp.int32(0))
    for j in range(7):
        pltpu.sync_copy(zrow_v.at[pl.ds(0, 80)], acc_sh.at[pl.ds(sid * ROWS_PW + j * 80, 80)])
    pltpu.sync_copy(zrow_v.at[pl.ds(0, 64)], acc_sh.at[pl.ds(sid * ROWS_PW + 560, 64)])

    @pl.when(sid == NS - 1)
    def _():
        pltpu.sync_copy(zrow_v.at[pl.ds(0, 16)], acc_sh.at[pl.ds(9984, 16)])

    plsc.subcore_barrier()

    base = _worker_edge_base(cid, sid)

    def chunk_body(i, _):
        pltpu.sync_copy(dst_hbm.at[pl.ds(base + i * CH, CH)], dst_v)
        pltpu.sync_copy(ones_v, acc_sh.at[dst_v], add=True)
        return _

    lax.fori_loop(0, NCHUNK, chunk_body, jnp.int32(0))
    plsc.subcore_barrier()

    pltpu.sync_copy(acc_sh.at[pl.ds(sid * ROWS_PW, ROWS_PW)],
                    out_hbm.at[cid, pl.ds(sid * ROWS_PW, ROWS_PW)])

    @pl.when(sid == NS - 1)
    def _():
        pltpu.sync_copy(acc_sh.at[pl.ds(9984, 16)], out_hbm.at[cid, pl.ds(9984, 16)])


def _sc_degree(dst):
    f = pl.kernel(
        _deg_body,
        out_type=jax.ShapeDtypeStruct((NC, N, H), jnp.float32),
        mesh=_mesh,
        compiler_params=_sc_params,
        scratch_types=[
            pltpu.VMEM_SHARED((N, H), jnp.float32),
            pltpu.VMEM((CH, H), jnp.float32),
            pltpu.VMEM((CH,), jnp.int32),
            pltpu.VMEM((CH, H), jnp.float32),
        ],
    )
    return f(dst)


def _agg_body(g_hbm, src_hbm, dst_hbm, out_hbm, acc_sh, src_v, dst_v, rows_v, sem):
    cid = lax.axis_index("c")
    sid = lax.axis_index("s")

    def zero_body(i, _):
        for j in range(8):
            rows_v[i, pl.ds(j * 16, 16)] = jnp.zeros((16,), jnp.float32)
        return _

    lax.fori_loop(0, CH, zero_body, jnp.int32(0))
    for j in range(7):
        pltpu.sync_copy(rows_v.at[pl.ds(0, 80)], acc_sh.at[pl.ds(sid * ROWS_PW + j * 80, 80)])
    pltpu.sync_copy(rows_v.at[pl.ds(0, 64)], acc_sh.at[pl.ds(sid * ROWS_PW + 560, 64)])

    @pl.when(sid == NS - 1)
    def _():
        pltpu.sync_copy(rows_v.at[pl.ds(0, 16)], acc_sh.at[pl.ds(9984, 16)])

    plsc.subcore_barrier()

    base = _worker_edge_base(cid, sid)

    def chunk_body(i, _):
        pltpu.sync_copy(src_hbm.at[pl.ds(base + i * CH, CH)], src_v)
        pltpu.sync_copy(dst_hbm.at[pl.ds(base + i * CH, CH)], dst_v)
        pltpu.async_copy(g_hbm.at[src_v], rows_v, sem).wait()
        pltpu.sync_copy(rows_v, acc_sh.at[dst_v], add=True)
        return _

    lax.fori_loop(0, NCHUNK, chunk_body, jnp.int32(0))
    plsc.subcore_barrier()

    pltpu.sync_copy(acc_sh.at[pl.ds(sid * ROWS_PW, ROWS_PW)],
                    out_hbm.at[cid, pl.ds(sid * ROWS_PW, ROWS_PW)])

    @pl.when(sid == NS - 1)
    def _():
        pltpu.sync_copy(acc_sh.at[pl.ds(9984, 16)], out_hbm.at[cid, pl.ds(9984, 16)])


def _sc_gcn_agg(g, src, dst):
    f = pl.kernel(
        _agg_body,
        out_type=jax.ShapeDtypeStruct((NC, N, H), jnp.float32),
        mesh=_mesh,
        compiler_params=_sc_params,
        scratch_types=[
            pltpu.VMEM_SHARED((N, H), jnp.float32),
            pltpu.VMEM((CH,), jnp.int32),
            pltpu.VMEM((CH,), jnp.int32),
            pltpu.VMEM((CH, H), jnp.float32),
            pltpu.SemaphoreType.DMA,
        ],
    )
    return f(g, src, dst)


def _pre_body(p_hbm, q_hbm, src_hbm, dst_hbm, out_hbm, src_v, dst_v, pa_v, qb_v, sem):
    cid = lax.axis_index("c")
    sid = lax.axis_index("s")
    base = _worker_edge_base(cid, sid)

    def chunk_body(i, _):
        pltpu.sync_copy(src_hbm.at[pl.ds(base + i * CH, CH)], src_v)
        pltpu.sync_copy(dst_hbm.at[pl.ds(base + i * CH, CH)], dst_v)
        pltpu.async_copy(p_hbm.at[dst_v], pa_v, sem).wait()
        pltpu.async_copy(q_hbm.at[src_v], qb_v, sem).wait()

        def row_body(r, _):
            for j in range(8):
                v = pa_v[r, pl.ds(j * 16, 16)] + qb_v[r, pl.ds(j * 16, 16)]
                pa_v[r, pl.ds(j * 16, 16)] = jnp.maximum(v, 0.0)
            return _

        lax.fori_loop(0, CH, row_body, jnp.int32(0))
        pltpu.sync_copy(pa_v, out_hbm.at[pl.ds(base + i * CH, CH)])
        return _

    lax.fori_loop(0, NCHUNK, chunk_body, jnp.int32(0))


def _sc_edge_pre(p, q, src, dst):
    f = pl.kernel(
        _pre_body,
        out_type=jax.ShapeDtypeStruct((E, H), jnp.float32),
        mesh=_mesh,
        compiler_params=_sc_params,
        scratch_types=[
            pltpu.VMEM((CH,), jnp.int32),
            pltpu.VMEM((CH,), jnp.int32),
            pltpu.VMEM((CH, H), jnp.float32),
            pltpu.VMEM((CH, H), jnp.float32),
            pltpu.SemaphoreType.DMA,
        ],
    )
    return f(p, q, src, dst)


SCH = 4000
NSCH = E // SCH
GCH = 128


def _smax_body(m_hbm, dst_hbm, out_hbm, dstb_v, ids_v, lds_v, rows_v, acc_v, sem):
    cid = lax.axis_index("c")
    sid = lax.axis_index("s")
    wid = sid * NC + cid
    lo = wid * OWN
    hi = jnp.where(wid == NW - 1, N, lo + OWN)
    nown = 328

    def init_body(i, _):
        for j in range(8):
            acc_v[i, pl.ds(j * 16, 16)] = jnp.full((16,), NEG, jnp.float32)
        return _

    lax.fori_loop(0, nown + 8, init_body, jnp.int32(0))

    iota = lax.broadcasted_iota(jnp.int32, (16,), 0)

    def outer_body(i, _):
        ebase = i * SCH
        pltpu.sync_copy(dst_hbm.at[pl.ds(ebase, SCH)], dstb_v)

        def scan_body(t, cnt):
            d16 = dstb_v[pl.ds(t * 16, 16)]
            msk = (d16 >= lo) & (d16 < hi)
            mi = msk.astype(jnp.int32)
            cs = plsc.cumsum(mi)
            pos = cnt + cs - mi
            plsc.store_scatter(ids_v, [pos], iota + (ebase + t * 16), mask=msk)
            plsc.store_scatter(lds_v, [pos], d16 - lo, mask=msk)
            return cnt + cs[15]

        cnt = lax.fori_loop(0, SCH // 16, scan_body, jnp.int32(0))

        for k in range(8):
            ids_v[pl.ds(cnt + k * 16, 16)] = jnp.full((16,), ebase, jnp.int32)
            lds_v[pl.ds(cnt + k * 16, 16)] = jnp.full((16,), nown, jnp.int32)

        nch = (cnt + GCH - 1) // GCH

        def gather_body(k, _):
            pltpu.async_copy(m_hbm.at[ids_v.at[pl.ds(k * GCH, GCH)]], rows_v, sem).wait()

            def row_body(r, _):
                ld = lds_v[pl.ds(k * GCH + r, 16)][0]
                for j in range(8):
                    cur = acc_v[ld, pl.ds(j * 16, 16)]
                    acc_v[ld, pl.ds(j * 16, 16)] = jnp.maximum(cur, rows_v[r, pl.ds(j * 16, 16)])
                return _

            lax.fori_loop(0, GCH, row_body, jnp.int32(0))
            return _

        lax.fori_loop(0, nch, gather_body, jnp.int32(0))
        return _

    lax.fori_loop(0, NSCH, outer_body, jnp.int32(0))

    pltpu.sync_copy(acc_v.at[pl.ds(0, OWN)], out_hbm.at[pl.ds(wid * OWN, OWN)])

    @pl.when(wid == NW - 1)
    def _():
        pltpu.sync_copy(acc_v.at[pl.ds(OWN, 16)], out_hbm.at[pl.ds(9984, 16)])


def _sc_segmax(m, dst):
    f = pl.kernel(
        _smax_body,
        out_type=jax.ShapeDtypeStruct((N, H), jnp.float32),
        mesh=_mesh,
        compiler_params=_sc_params,
        scratch_types=[
            pltpu.VMEM((SCH,), jnp.int32),
            pltpu.VMEM((SCH + GCH + 16,), jnp.int32),
            pltpu.VMEM((SCH + GCH + 16,), jnp.int32),
            pltpu.VMEM((GCH, H), jnp.float32),
            pltpu.VMEM((344, H), jnp.float32),
            pltpu.SemaphoreType.DMA,
        ],
    )
    return f(m, dst)


BN = 2000
GN = N // BN
BE = 4000
GE = E // BE


def _dinv_from_deg(degp):
    deg = degp[0, :, 0:1] + degp[1, :, 0:1] + 1.0
    return lax.rsqrt(jnp.maximum(deg, 1.0))


def _enc_body(x_ref, ew_ref, eb_ref, lg_ref, lb_ref, w1_ref, degp_ref, g_ref, hw_ref):
    h = jnp.dot(x_ref[...], ew_ref[...], preferred_element_type=jnp.float32) + eb_ref[...]
    h = jnp.maximum(h, 0.0)
    mu = jnp.mean(h, axis=-1, keepdims=True)
    var = jnp.mean((h - mu) ** 2, axis=-1, keepdims=True)
    h = lg_ref[...] * (h - mu) * lax.rsqrt(var + 1e-5) + lb_ref[...]
    dinv = _dinv_from_deg(degp_ref[...])
    hw = jnp.dot(h, w1_ref[...], preferred_element_type=jnp.float32)
    hw_ref[...] = hw
    g_ref[...] = dinv * hw


def _tc_encoder(x, enc_W, enc_b, ln_g, ln_b, W1, degp):
    wspec = pl.BlockSpec((H, H), lambda i: (0, 0))
    vspec = pl.BlockSpec((1, H), lambda i: (0, 0))
    nspec = pl.BlockSpec((BN, H), lambda i: (i, 0))
    dspec = pl.BlockSpec((NC, BN, H), lambda i: (0, i, 0))
    return pl.pallas_call(
        _enc_body,
        grid=(GN,),
        in_specs=[nspec, wspec, vspec, vspec, vspec, wspec, dspec],
        out_specs=[nspec, nspec],
        out_shape=[jax.ShapeDtypeStruct((N, H), jnp.float32)] * 2,
    )(x, enc_W, enc_b, ln_g, ln_b, W1, degp)


def _mid_body(aggp_ref, hw_ref, degp_ref, b_ref, wn_ref, g_ref, hwn_ref):
    dinv = _dinv_from_deg(degp_ref[...])
    agg = aggp_ref[0] + aggp_ref[1]
    h = jnp.maximum(dinv * agg + dinv * dinv * hw_ref[...] + b_ref[...], 0.0)
    hwn = jnp.dot(h, wn_ref[...], preferred_element_type=jnp.float32)
    hwn_ref[...] = hwn
    g_ref[...] = dinv * hwn


def _tc_gcn_mid(aggp, hw, degp, b, Wn):
    wspec = pl.BlockSpec((H, H), lambda i: (0, 0))
    vspec = pl.BlockSpec((1, H), lambda i: (0, 0))
    nspec = pl.BlockSpec((BN, H), lambda i: (i, 0))
    aspec = pl.BlockSpec((NC, BN, H), lambda i: (0, i, 0))
    dspec = pl.BlockSpec((NC, BN, H), lambda i: (0, i, 0))
    return pl.pallas_call(
        _mid_body,
        grid=(GN,),
        in_specs=[aspec, nspec, dspec, vspec, wspec],
        out_specs=[nspec, nspec],
        out_shape=[jax.ShapeDtypeStruct((N, H), jnp.float32)] * 2,
    )(aggp, hw, degp, b, Wn)


def _fin_body(aggp_ref, hw_ref, degp_ref, b_ref, ecw1_ref, ecb1_ref, p_ref, q_ref):
    dinv = _dinv_from_deg(degp_ref[...])
    agg = aggp_ref[0] + aggp_ref[1]
    h = jnp.maximum(dinv * agg + dinv * dinv * hw_ref[...] + b_ref[...], 0.0)
    wa = ecw1_ref[0:H, :]
    wb = ecw1_ref[H:2 * H, :]
    p_ref[...] = jnp.dot(h, wa - wb, preferred_element_type=jnp.float32) + ecb1_ref[...]
    q_ref[...] = jnp.dot(h, wb, preferred_element_type=jnp.float32)


def _tc_gcn_fin(aggp, hw, degp, b, ec_W1, ec_b1):
    vspec = pl.BlockSpec((1, H), lambda i: (0, 0))
    nspec = pl.BlockSpec((BN, H), lambda i: (i, 0))
    aspec = pl.BlockSpec((NC, BN, H), lambda i: (0, i, 0))
    dspec = pl.BlockSpec((NC, BN, H), lambda i: (0, i, 0))
    w2spec = pl.BlockSpec((2 * H, H), lambda i: (0, 0))
    return pl.pallas_call(
        _fin_body,
        grid=(GN,),
        in_specs=[aspec, nspec, dspec, vspec, w2spec, vspec],
        out_specs=[nspec, nspec],
        out_shape=[jax.ShapeDtypeStruct((N, H), jnp.float32)] * 2,
    )(aggp, hw, degp, b, ec_W1, ec_b1)


def _msg_body(pre_ref, w2_ref, b2_ref, m_ref):
    m_ref[...] = jnp.dot(pre_ref[...], w2_ref[...], preferred_element_type=jnp.float32) + b2_ref[...]


def _tc_edge_msg(pre, ec_W2, ec_b2):
    espec = pl.BlockSpec((BE, H), lambda i: (i, 0))
    wspec = pl.BlockSpec((H, H), lambda i: (0, 0))
    vspec = pl.BlockSpec((1, H), lambda i: (0, 0))
    return pl.pallas_call(
        _msg_body,
        grid=(GE,),
        in_specs=[espec, wspec, vspec],
        out_specs=espec,
        out_shape=jax.ShapeDtypeStruct((E, H), jnp.float32),
    )(pre, ec_W2, ec_b2)


def _head_body(sm_ref, pw1_ref, pb1_ref, cw1_ref, cb1_ref, wa_ref, wb_ref, bias_ref, y_ref):
    sm = sm_ref[...]
    h = jnp.where(sm > NEG, sm, 0.0)
    t1 = jnp.maximum(jnp.dot(h, pw1_ref[...], preferred_element_type=jnp.float32) + pb1_ref[...], 0.0)
    t2 = jnp.maximum(jnp.dot(h, cw1_ref[...], preferred_element_type=jnp.float32) + cb1_ref[...], 0.0)
    y = (jnp.dot(t1, wa_ref[...], preferred_element_type=jnp.float32)
         + jnp.dot(t2, wb_ref[...], preferred_element_type=jnp.float32) + bias_ref[...])
    col = lax.broadcasted_iota(jnp.int32, y.shape, 1)
    y_ref[...] = jnp.where(col == 2, jax.nn.sigmoid(y), y)


def _tc_heads(sm, ph_W1, ph_b1, ch_W1, ch_b1, wa, wb, bias):
    wspec = pl.BlockSpec((H, H), lambda i: (0, 0))
    vspec = pl.BlockSpec((1, H), lambda i: (0, 0))
    nspec = pl.BlockSpec((BN, H), lambda i: (i, 0))
    return pl.pallas_call(
        _head_body,
        grid=(GN,),
        in_specs=[nspec, wspec, vspec, wspec, vspec, wspec, wspec, vspec],
        out_specs=nspec,
        out_shape=jax.ShapeDtypeStruct((N, H), jnp.float32),
    )(sm, ph_W1, ph_b1, ch_W1, ch_b1, wa, wb, bias)


def kernel(x, edge_index, enc_W, enc_b, ln_g, ln_b, W1, b1, W2, b2, W3, b3,
           ec_W1, ec_b1, ec_W2, ec_b2, ph_W1, ph_b1, ph_W2, ph_b2,
           ch_W1, ch_b1, ch_W2, ch_b2):
    src = edge_index[0]
    dst = edge_index[1]

    degp = _sc_degree(dst)

    row = lambda v: v.reshape(1, H)
    g, hw = _tc_encoder(x, enc_W, row(enc_b), row(ln_g), row(ln_b), W1, degp)

    aggp = _sc_gcn_agg(g, src, dst)
    g, hw = _tc_gcn_mid(aggp, hw, degp, row(b1), W2)
    aggp = _sc_gcn_agg(g, src, dst)
    g, hw = _tc_gcn_mid(aggp, hw, degp, row(b2), W3)
    aggp = _sc_gcn_agg(g, src, dst)
    p, q = _tc_gcn_fin(aggp, hw, degp, row(b3), ec_W1, ec_b1.reshape(1, H))

    pre = _sc_edge_pre(p, q, src, dst)
    m = _tc_edge_msg(pre, ec_W2, ec_b2.reshape(1, H))
    sm = _sc_segmax(m, dst)

    zpad = jnp.zeros((H, H - 3), jnp.float32)
    wa = jnp.concatenate([ph_W2, jnp.zeros((H, 1), jnp.float32), zpad], axis=1)
    wb = jnp.concatenate([jnp.zeros((H, 2), jnp.float32), ch_W2, zpad], axis=1)
    bias = jnp.concatenate([ph_b2, ch_b2, jnp.zeros((H - 3,), jnp.float32)]).reshape(1, H)

    y = _tc_heads(sm, ph_W1, ph_b1.reshape(1, H), ch_W1, ch_b1.reshape(1, H), wa, wb, bias)
    return y[:, :3]


# speedup vs baseline: 4.7507x; 1.1569x over previous
import functools
import jax
import jax.numpy as jnp
from jax import lax
from jax.experimental import pallas as pl
from jax.experimental.pallas import tpu as pltpu
from jax.experimental.pallas import tpu_sc as plsc

N = 10000
E = 320000
H = 128
NC, NS = 2, 16
NW = NC * NS
EPC = E // NC
EPW = E // NW
CH = 80
NCHUNK = EPW // CH
ROWS_PW = 624
OWN = 312
NEG = -3.0e38

_mesh = plsc.VectorSubcoreMesh(core_axis_name="c", subcore_axis_name="s")
_sc_params = pltpu.CompilerParams(needs_layout_passes=False)


def _worker_edge_base(cid, sid):
    return cid * EPC + sid * EPW


def _deg_body(dst_hbm, out_hbm, acc_sh, ones_v, dst_v, zrow_v):
    cid = lax.axis_index("c")
    sid = lax.axis_index("s")

    def fill_body(i, _):
        for j in range(8):
            ones_v[i, pl.ds(j * 16, 16)] = jnp.ones((16,), jnp.float32)
            zrow_v[i, pl.ds(j * 16, 16)] = jnp.zeros((16,), jnp.float32)
        return _

    lax.fori_loop(0, CH, fill_body, jnp.int32(0))
    for j in range(7):
        pltpu.sync_copy(zrow_v.at[pl.ds(0, 80)], acc_sh.at[pl.ds(sid * ROWS_PW + j * 80, 80)])
    pltpu.sync_copy(zrow_v.at[pl.ds(0, 64)], acc_sh.at[pl.ds(sid * ROWS_PW + 560, 64)])

    @pl.when(sid == NS - 1)
    def _():
        pltpu.sync_copy(zrow_v.at[pl.ds(0, 16)], acc_sh.at[pl.ds(9984, 16)])

    plsc.subcore_barrier()

    base = _worker_edge_base(cid, sid)

    def chunk_body(i, _):
        pltpu.sync_copy(dst_hbm.at[pl.ds(base + i * CH, CH)], dst_v)
        pltpu.sync_copy(ones_v, acc_sh.at[dst_v], add=True)
        return _

    lax.fori_loop(0, NCHUNK, chunk_body, jnp.int32(0))
    plsc.subcore_barrier()

    pltpu.sync_copy(acc_sh.at[pl.ds(sid * ROWS_PW, ROWS_PW)],
                    out_hbm.at[cid, pl.ds(sid * ROWS_PW, ROWS_PW)])

    @pl.when(sid == NS - 1)
    def _():
        pltpu.sync_copy(acc_sh.at[pl.ds(9984, 16)], out_hbm.at[cid, pl.ds(9984, 16)])


def _sc_degree(dst):
    f = pl.kernel(
        _deg_body,
        out_type=jax.ShapeDtypeStruct((NC, N, H), jnp.float32),
        mesh=_mesh,
        compiler_params=_sc_params,
        scratch_types=[
            pltpu.VMEM_SHARED((N, H), jnp.float32),
            pltpu.VMEM((CH, H), jnp.float32),
            pltpu.VMEM((CH,), jnp.int32),
            pltpu.VMEM((CH, H), jnp.float32),
        ],
    )
    return f(dst)


def _agg_body(g_hbm, src_hbm, dst2_hbm, out_hbm, acc_sh, srcst_v, dstst_v,
              rows0_v, rows1_v, sem0, sem1):
    cid = lax.axis_index("c")
    sid = lax.axis_index("s")

    def zero_body(i, _):
        for j in range(8):
            rows0_v[i, pl.ds(j * 16, 16)] = jnp.zeros((16,), jnp.float32)
        return _

    lax.fori_loop(0, CH, zero_body, jnp.int32(0))
    for j in range(7):
        pltpu.sync_copy(rows0_v.at[pl.ds(0, 80)], acc_sh.at[pl.ds(sid * ROWS_PW + j * 80, 80)])
    pltpu.sync_copy(rows0_v.at[pl.ds(0, 64)], acc_sh.at[pl.ds(sid * ROWS_PW + 560, 64)])

    @pl.when(sid == NS - 1)
    def _():
        pltpu.sync_copy(rows0_v.at[pl.ds(0, 16)], acc_sh.at[pl.ds(9984, 16)])

    base = _worker_edge_base(cid, sid)
    wk = cid * NS + sid
    pltpu.sync_copy(src_hbm.at[pl.ds(base, EPW)], srcst_v)
    pltpu.sync_copy(dst2_hbm.at[wk], dstst_v)
    plsc.subcore_barrier()

    cp0 = pltpu.async_copy(g_hbm.at[srcst_v.at[pl.ds(0, CH)]], rows0_v, sem0)

    def pair_body(j, car):
        i0 = 2 * j
        cp0 = pltpu.make_async_copy(g_hbm.at[srcst_v.at[pl.ds(i0 * CH, CH)]], rows0_v, sem0)
        cp0.wait()
        pltpu.async_copy(g_hbm.at[srcst_v.at[pl.ds((i0 + 1) * CH, CH)]], rows1_v, sem1)
        pltpu.sync_copy(rows0_v, acc_sh.at[dstst_v.at[i0]], add=True)
        pltpu.make_async_copy(g_hbm.at[srcst_v.at[pl.ds((i0 + 1) * CH, CH)]], rows1_v, sem1).wait()

        @pl.when(i0 + 2 < NCHUNK)
        def _():
            pltpu.async_copy(g_hbm.at[srcst_v.at[pl.ds((i0 + 2) * CH, CH)]], rows0_v, sem0)

        pltpu.sync_copy(rows1_v, acc_sh.at[dstst_v.at[i0 + 1]], add=True)
        return car

    lax.fori_loop(0, NCHUNK // 2, pair_body, jnp.int32(0))
    pltpu.make_async_copy(g_hbm.at[srcst_v.at[pl.ds((NCHUNK - 1) * CH, CH)]], rows0_v, sem0).wait()
    pltpu.sync_copy(rows0_v, acc_sh.at[dstst_v.at[NCHUNK - 1]], add=True)
    plsc.subcore_barrier()

    pltpu.sync_copy(acc_sh.at[pl.ds(sid * ROWS_PW, ROWS_PW)],
                    out_hbm.at[cid, pl.ds(sid * ROWS_PW, ROWS_PW)])

    @pl.when(sid == NS - 1)
    def _():
        pltpu.sync_copy(acc_sh.at[pl.ds(9984, 16)], out_hbm.at[cid, pl.ds(9984, 16)])


def _sc_gcn_agg(g, src, dst2):
    f = pl.kernel(
        _agg_body,
        out_type=jax.ShapeDtypeStruct((NC, N, H), jnp.float32),
        mesh=_mesh,
        compiler_params=_sc_params,
        scratch_types=[
            pltpu.VMEM_SHARED((N, H), jnp.float32),
            pltpu.VMEM((EPW,), jnp.int32),
            pltpu.VMEM((NCHUNK, CH), jnp.int32),
            pltpu.VMEM((CH, H), jnp.float32),
            pltpu.VMEM((CH, H), jnp.float32),
            pltpu.SemaphoreType.DMA,
            pltpu.SemaphoreType.DMA,
        ],
    )
    return f(g, src, dst2)


def _pre_body(p_hbm, q_hbm, src_hbm, dst_hbm, out_hbm, src_v, dst_v, pa_v, qb_v, sem):
    cid = lax.axis_index("c")
    sid = lax.axis_index("s")
    base = _worker_edge_base(cid, sid)

    def chunk_body(i, _):
        pltpu.sync_copy(src_hbm.at[pl.ds(base + i * CH, CH)], src_v)
        pltpu.sync_copy(dst_hbm.at[pl.ds(base + i * CH, CH)], dst_v)
        pltpu.async_copy(p_hbm.at[dst_v], pa_v, sem).wait()
        pltpu.async_copy(q_hbm.at[src_v], qb_v, sem).wait()

        def row_body(r, _):
            for j in range(8):
                v = pa_v[r, pl.ds(j * 16, 16)] + qb_v[r, pl.ds(j * 16, 16)]
                pa_v[r, pl.ds(j * 16, 16)] = jnp.maximum(v, 0.0)
            return _

        lax.fori_loop(0, CH, row_body, jnp.int32(0))
        pltpu.sync_copy(pa_v, out_hbm.at[pl.ds(base + i * CH, CH)])
        return _

    lax.fori_loop(0, NCHUNK, chunk_body, jnp.int32(0))


def _sc_edge_pre(p, q, src, dst):
    f = pl.kernel(
        _pre_body,
        out_type=jax.ShapeDtypeStruct((E, H), jnp.float32),
        mesh=_mesh,
        compiler_params=_sc_params,
        scratch_types=[
            pltpu.VMEM((CH,), jnp.int32),
            pltpu.VMEM((CH,), jnp.int32),
            pltpu.VMEM((CH, H), jnp.float32),
            pltpu.VMEM((CH, H), jnp.float32),
            pltpu.SemaphoreType.DMA,
        ],
    )
    return f(p, q, src, dst)


SCH = 4000
NSCH = E // SCH
GCH = 128


def _smax_body(m_hbm, dst_hbm, out_hbm, dstb_v, ids_v, lds_v, rows_v, acc_v, sem):
    cid = lax.axis_index("c")
    sid = lax.axis_index("s")
    wid = sid * NC + cid
    lo = wid * OWN
    hi = jnp.where(wid == NW - 1, N, lo + OWN)
    nown = 328

    def init_body(i, _):
        for j in range(8):
            acc_v[i, pl.ds(j * 16, 16)] = jnp.full((16,), NEG, jnp.float32)
        return _

    lax.fori_loop(0, nown + 8, init_body, jnp.int32(0))

    iota = lax.broadcasted_iota(jnp.int32, (16,), 0)

    def outer_body(i, _):
        ebase = i * SCH
        pltpu.sync_copy(dst_hbm.at[pl.ds(ebase, SCH)], dstb_v)

        def scan_body(t, cnt):
            d16 = dstb_v[pl.ds(t * 16, 16)]
            msk = (d16 >= lo) & (d16 < hi)
            mi = msk.astype(jnp.int32)
            cs = plsc.cumsum(mi)
            pos = cnt + cs - mi
            plsc.store_scatter(ids_v, [pos], iota + (ebase + t * 16), mask=msk)
            plsc.store_scatter(lds_v, [pos], d16 - lo, mask=msk)
            return cnt + cs[15]

        cnt = lax.fori_loop(0, SCH // 16, scan_body, jnp.int32(0))

        for k in range(8):
            ids_v[pl.ds(cnt + k * 16, 16)] = jnp.full((16,), ebase, jnp.int32)
            lds_v[pl.ds(cnt + k * 16, 16)] = jnp.full((16,), nown, jnp.int32)

        nch = (cnt + GCH - 1) // GCH

        def gather_body(k, _):
            pltpu.async_copy(m_hbm.at[ids_v.at[pl.ds(k * GCH, GCH)]], rows_v, sem).wait()

            def row_body(r, _):
                ld = lds_v[pl.ds(k * GCH + r, 16)][0]
                for j in range(8):
                    cur = acc_v[ld, pl.ds(j * 16, 16)]
                    acc_v[ld, pl.ds(j * 16, 16)] = jnp.maximum(cur, rows_v[r, pl.ds(j * 16, 16)])
                return _

            lax.fori_loop(0, GCH, row_body, jnp.int32(0))
            return _

        lax.fori_loop(0, nch, gather_body, jnp.int32(0))
        return _

    lax.fori_loop(0, NSCH, outer_body, jnp.int32(0))

    pltpu.sync_copy(acc_v.at[pl.ds(0, OWN)], out_hbm.at[pl.ds(wid * OWN, OWN)])

    @pl.when(wid == NW - 1)
    def _():
        pltpu.sync_copy(acc_v.at[pl.ds(OWN, 16)], out_hbm.at[pl.ds(9984, 16)])


def _sc_segmax(m, dst):
    f = pl.kernel(
        _smax_body,
        out_type=jax.ShapeDtypeStruct((N, H), jnp.float32),
        mesh=_mesh,
        compiler_params=_sc_params,
        scratch_types=[
            pltpu.VMEM((SCH,), jnp.int32),
            pltpu.VMEM((SCH + GCH + 16,), jnp.int32),
            pltpu.VMEM((SCH + GCH + 16,), jnp.int32),
            pltpu.VMEM((GCH, H), jnp.float32),
            pltpu.VMEM((344, H), jnp.float32),
            pltpu.SemaphoreType.DMA,
        ],
    )
    return f(m, dst)


BN = 2000
GN = N // BN
BE = 4000
GE = E // BE


def _dinv_from_deg(degp):
    deg = degp[0, :, 0:1] + degp[1, :, 0:1] + 1.0
    return lax.rsqrt(jnp.maximum(deg, 1.0))


def _enc_body(x_ref, ew_ref, eb_ref, lg_ref, lb_ref, w1_ref, degp_ref, g_ref, hw_ref):
    h = jnp.dot(x_ref[...], ew_ref[...], preferred_element_type=jnp.float32) + eb_ref[...]
    h = jnp.maximum(h, 0.0)
    mu = jnp.mean(h, axis=-1, keepdims=True)
    var = jnp.mean((h - mu) ** 2, axis=-1, keepdims=True)
    h = lg_ref[...] * (h - mu) * lax.rsqrt(var + 1e-5) + lb_ref[...]
    dinv = _dinv_from_deg(degp_ref[...])
    hw = jnp.dot(h, w1_ref[...], preferred_element_type=jnp.float32)
    hw_ref[...] = hw
    g_ref[...] = dinv * hw


def _tc_encoder(x, enc_W, enc_b, ln_g, ln_b, W1, degp):
    wspec = pl.BlockSpec((H, H), lambda i: (0, 0))
    vspec = pl.BlockSpec((1, H), lambda i: (0, 0))
    nspec = pl.BlockSpec((BN, H), lambda i: (i, 0))
    dspec = pl.BlockSpec((NC, BN, H), lambda i: (0, i, 0))
    return pl.pallas_call(
        _enc_body,
        grid=(GN,),
        in_specs=[nspec, wspec, vspec, vspec, vspec, wspec, dspec],
        out_specs=[nspec, nspec],
        out_shape=[jax.ShapeDtypeStruct((N, H), jnp.float32)] * 2,
    )(x, enc_W, enc_b, ln_g, ln_b, W1, degp)


def _mid_body(aggp_ref, hw_ref, degp_ref, b_ref, wn_ref, g_ref, hwn_ref):
    dinv = _dinv_from_deg(degp_ref[...])
    agg = aggp_ref[0] + aggp_ref[1]
    h = jnp.maximum(dinv * agg + dinv * dinv * hw_ref[...] + b_ref[...], 0.0)
    hwn = jnp.dot(h, wn_ref[...], preferred_element_type=jnp.float32)
    hwn_ref[...] = hwn
    g_ref[...] = dinv * hwn


def _tc_gcn_mid(aggp, hw, degp, b, Wn):
    wspec = pl.BlockSpec((H, H), lambda i: (0, 0))
    vspec = pl.BlockSpec((1, H), lambda i: (0, 0))
    nspec = pl.BlockSpec((BN, H), lambda i: (i, 0))
    aspec = pl.BlockSpec((NC, BN, H), lambda i: (0, i, 0))
    dspec = pl.BlockSpec((NC, BN, H), lambda i: (0, i, 0))
    return pl.pallas_call(
        _mid_body,
        grid=(GN,),
        in_specs=[aspec, nspec, dspec, vspec, wspec],
        out_specs=[nspec, nspec],
        out_shape=[jax.ShapeDtypeStruct((N, H), jnp.float32)] * 2,
    )(aggp, hw, degp, b, Wn)


def _fin_body(aggp_ref, hw_ref, degp_ref, b_ref, ecw1_ref, ecb1_ref, p_ref, q_ref):
    dinv = _dinv_from_deg(degp_ref[...])
    agg = aggp_ref[0] + aggp_ref[1]
    h = jnp.maximum(dinv * agg + dinv * dinv * hw_ref[...] + b_ref[...], 0.0)
    wa = ecw1_ref[0:H, :]
    wb = ecw1_ref[H:2 * H, :]
    p_ref[...] = jnp.dot(h, wa - wb, preferred_element_type=jnp.float32) + ecb1_ref[...]
    q_ref[...] = jnp.dot(h, wb, preferred_element_type=jnp.float32)


def _tc_gcn_fin(aggp, hw, degp, b, ec_W1, ec_b1):
    vspec = pl.BlockSpec((1, H), lambda i: (0, 0))
    nspec = pl.BlockSpec((BN, H), lambda i: (i, 0))
    aspec = pl.BlockSpec((NC, BN, H), lambda i: (0, i, 0))
    dspec = pl.BlockSpec((NC, BN, H), lambda i: (0, i, 0))
    w2spec = pl.BlockSpec((2 * H, H), lambda i: (0, 0))
    return pl.pallas_call(
        _fin_body,
        grid=(GN,),
        in_specs=[aspec, nspec, dspec, vspec, w2spec, vspec],
        out_specs=[nspec, nspec],
        out_shape=[jax.ShapeDtypeStruct((N, H), jnp.float32)] * 2,
    )(aggp, hw, degp, b, ec_W1, ec_b1)


def _msg_body(pre_ref, w2_ref, b2_ref, m_ref):
    m_ref[...] = jnp.dot(pre_ref[...], w2_ref[...], preferred_element_type=jnp.float32) + b2_ref[...]


def _tc_edge_msg(pre, ec_W2, ec_b2):
    espec = pl.BlockSpec((BE, H), lambda i: (i, 0))
    wspec = pl.BlockSpec((H, H), lambda i: (0, 0))
    vspec = pl.BlockSpec((1, H), lambda i: (0, 0))
    return pl.pallas_call(
        _msg_body,
        grid=(GE,),
        in_specs=[espec, wspec, vspec],
        out_specs=espec,
        out_shape=jax.ShapeDtypeStruct((E, H), jnp.float32),
    )(pre, ec_W2, ec_b2)


def _head_body(sm_ref, pw1_ref, pb1_ref, cw1_ref, cb1_ref, wa_ref, wb_ref, bias_ref, y_ref):
    sm = sm_ref[...]
    h = jnp.where(sm > NEG, sm, 0.0)
    t1 = jnp.maximum(jnp.dot(h, pw1_ref[...], preferred_element_type=jnp.float32) + pb1_ref[...], 0.0)
    t2 = jnp.maximum(jnp.dot(h, cw1_ref[...], preferred_element_type=jnp.float32) + cb1_ref[...], 0.0)
    y = (jnp.dot(t1, wa_ref[...], preferred_element_type=jnp.float32)
         + jnp.dot(t2, wb_ref[...], preferred_element_type=jnp.float32) + bias_ref[...])
    col = lax.broadcasted_iota(jnp.int32, y.shape, 1)
    y_ref[...] = jnp.where(col == 2, jax.nn.sigmoid(y), y)


def _tc_heads(sm, ph_W1, ph_b1, ch_W1, ch_b1, wa, wb, bias):
    wspec = pl.BlockSpec((H, H), lambda i: (0, 0))
    vspec = pl.BlockSpec((1, H), lambda i: (0, 0))
    nspec = pl.BlockSpec((BN, H), lambda i: (i, 0))
    return pl.pallas_call(
        _head_body,
        grid=(GN,),
        in_specs=[nspec, wspec, vspec, wspec, vspec, wspec, wspec, vspec],
        out_specs=nspec,
        out_shape=jax.ShapeDtypeStruct((N, H), jnp.float32),
    )(sm, ph_W1, ph_b1, ch_W1, ch_b1, wa, wb, bias)


def kernel(x, edge_index, enc_W, enc_b, ln_g, ln_b, W1, b1, W2, b2, W3, b3,
           ec_W1, ec_b1, ec_W2, ec_b2, ph_W1, ph_b1, ph_W2, ph_b2,
           ch_W1, ch_b1, ch_W2, ch_b2):
    src = edge_index[0]
    dst = edge_index[1]

    dst3 = dst.reshape(NW, NCHUNK, CH)

    degp = _sc_degree(dst)

    row = lambda v: v.reshape(1, H)
    g, hw = _tc_encoder(x, enc_W, row(enc_b), row(ln_g), row(ln_b), W1, degp)

    aggp = _sc_gcn_agg(g, src, dst3)
    g, hw = _tc_gcn_mid(aggp, hw, degp, row(b1), W2)
    aggp = _sc_gcn_agg(g, src, dst3)
    g, hw = _tc_gcn_mid(aggp, hw, degp, row(b2), W3)
    aggp = _sc_gcn_agg(g, src, dst3)
    p, q = _tc_gcn_fin(aggp, hw, degp, row(b3), ec_W1, ec_b1.reshape(1, H))

    pre = _sc_edge_pre(p, q, src, dst)
    m = _tc_edge_msg(pre, ec_W2, ec_b2.reshape(1, H))
    sm = _sc_segmax(m, dst)

    zpad = jnp.zeros((H, H - 3), jnp.float32)
    wa = jnp.concatenate([ph_W2, jnp.zeros((H, 1), jnp.float32), zpad], axis=1)
    wb = jnp.concatenate([jnp.zeros((H, 2), jnp.float32), ch_W2, zpad], axis=1)
    bias = jnp.concatenate([ph_b2, ch_b2, jnp.zeros((H - 3,), jnp.float32)]).reshape(1, H)

    y = _tc_heads(sm, ph_W1, ph_b1.reshape(1, H), ch_W1, ch_b1.reshape(1, H), wa, wb, bias)
    return y[:, :3]
